# baseline (speedup 1.0000x reference)
import functools

import jax
import jax.numpy as jnp
from jax import lax
from jax.experimental import pallas as pl
from jax.experimental.pallas import tpu as pltpu
from jax.experimental.pallas import tpu_sc as plsc

B = 16384
N_DENSE = 13
N_SPARSE = 26
VOCAB = 100000
D = 16
NF = N_DENSE + N_SPARSE

NC = 2
NS = 16
NW = NC * NS
ROWS_PER_W = B // NW
CHUNK = 128
N_CHUNKS = ROWS_PER_W // CHUNK
LANES = 16
VEC_PER_CHUNK = CHUNK * N_SPARSE // LANES


def _sc_gather(x_flat, tflat):
    mesh = plsc.VectorSubcoreMesh(core_axis_name="c", subcore_axis_name="s",
                                  num_cores=NC, num_subcores=NS)

    @functools.partial(
        pl.kernel,
        out_type=jax.ShapeDtypeStruct((B * N_SPARSE, D), jnp.float32),
        mesh=mesh,
        compiler_params=pltpu.CompilerParams(needs_layout_passes=False,
                                             use_tc_tiling_on_sc=False),
        scratch_types=[
            pltpu.VMEM((CHUNK * NF,), jnp.float32),
            pltpu.VMEM((CHUNK * N_SPARSE,), jnp.int32),
            pltpu.VMEM((CHUNK * N_SPARSE, D), jnp.float32),
            pltpu.SemaphoreType.DMA,
        ],
    )
    def k(x_hbm, tflat_hbm, g_hbm, xbuf, idxbuf, rows, sem):
        wid = lax.axis_index("s") * NC + lax.axis_index("c")
        base_w = wid * ROWS_PER_W

        def do_chunk(c, _):
            base = base_w + c * CHUNK
            pltpu.sync_copy(x_hbm.at[pl.ds(base * NF, CHUNK * NF)], xbuf)

            def conv(v, _):
                p = lax.iota(jnp.int32, LANES) + v * LANES
                r = lax.div(p, N_SPARSE)
                f = lax.rem(p, N_SPARSE)
                val = plsc.load_gather(xbuf, [r * NF + (f + N_DENSE)])
                idx = val.astype(jnp.int32) + f * VOCAB
                idxbuf[pl.ds(v * LANES, LANES)] = idx
                return _

            lax.fori_loop(0, VEC_PER_CHUNK, conv, None, unroll=4)
            pltpu.async_copy(tflat_hbm.at[idxbuf], rows, sem).wait()
            pltpu.sync_copy(rows, g_hbm.at[pl.ds(base * N_SPARSE,
                                                 CHUNK * N_SPARSE)])
            return _

        lax.fori_loop(0, N_CHUNKS, do_chunk, None)

    return k(x_flat, tflat)


def _tc_dense_ln(x, g2, e_w, e_b, sel, selt, gamma_f, beta_f):
    R = 512
    w208 = N_DENSE * D
    w624 = NF * D

    def body(x_ref, g_ref, ew_ref, eb_ref, s_ref, st_ref, gm_ref, bt_ref, o_ref):
        xd = x_ref[:, :N_DENSE]
        dense = jnp.dot(xd, ew_ref[...], preferred_element_type=jnp.float32)
        dense = dense + eb_ref[...]
        comb = jnp.concatenate([dense, g_ref[...]], axis=1)
        ssum = jnp.dot(comb, s_ref[...], preferred_element_type=jnp.float32)
        mean = ssum * (1.0 / D)
        cent = comb - jnp.dot(mean, st_ref[...], preferred_element_type=jnp.float32)
        vsum = jnp.dot(cent * cent, s_ref[...], preferred_element_type=jnp.float32)
        rinv = lax.rsqrt(vsum * (1.0 / D) + 1e-5)
        rb = jnp.dot(rinv, st_ref[...], preferred_element_type=jnp.float32)
        o_ref[...] = cent * rb * gm_ref[...] + bt_ref[...]

    full = lambda s: pl.BlockSpec(s, lambda i: (0, 0))
    return pl.pallas_call(
        body,
        grid=(B // R,),
        in_specs=[
            pl.BlockSpec((R, NF), lambda i: (i, 0)),
            pl.BlockSpec((R, N_SPARSE * D), lambda i: (i, 0)),
            full((N_DENSE, w208)),
            full((1, w208)),
            full((w624, NF)),
            full((NF, w624)),
            full((1, w624)),
            full((1, w624)),
        ],
        out_specs=pl.BlockSpec((R, w624), lambda i: (i, 0)),
        out_shape=jax.ShapeDtypeStruct((B, w624), jnp.float32),
    )(x, g2, e_w, e_b, sel, selt, gamma_f, beta_f)


def kernel(x, tables, lin_w, lin_b, ln_gamma, ln_beta):
    tflat = tables.reshape(N_SPARSE * VOCAB, D)
    g = _sc_gather(x.reshape(-1), tflat)

    e_w = (jnp.eye(N_DENSE, dtype=jnp.float32)[:, :, None]
           * lin_w[None, :, :]).reshape(N_DENSE, N_DENSE * D)
    e_b = lin_b.reshape(1, N_DENSE * D)
    sel = jnp.repeat(jnp.eye(NF, dtype=jnp.float32), D, axis=0)
    selt = sel.T
    gamma_f = jnp.tile(ln_gamma, NF).reshape(1, NF * D)
    beta_f = jnp.tile(ln_beta, NF).reshape(1, NF * D)

    out = _tc_dense_ln(x, g.reshape(B, N_SPARSE * D),
                       e_w, e_b, sel, selt, gamma_f, beta_f)
    return out.reshape(B, NF, D)


# speedup vs baseline: 1.3460x; 1.3460x over previous
import functools

import jax
import jax.numpy as jnp
from jax import lax
from jax.experimental import pallas as pl
from jax.experimental.pallas import tpu as pltpu
from jax.experimental.pallas import tpu_sc as plsc

B = 16384
N_DENSE = 13
N_SPARSE = 26
VOCAB = 100000
D = 16
NF = N_DENSE + N_SPARSE

NC = 2
NS = 16
NW = NC * NS
ROWS_PER_W = B // NW
LANES = 16


def _sc_gather(x_flat, tflat):
    mesh = plsc.VectorSubcoreMesh(core_axis_name="c", subcore_axis_name="s",
                                  num_cores=NC, num_subcores=NS)

    C = 16
    NCH = ROWS_PER_W // C
    NGR = C * N_SPARSE * D
    NTOK = C * N_SPARSE
    VGR = VOCAB // D

    @functools.partial(
        pl.kernel,
        out_type=jax.ShapeDtypeStruct((B * N_SPARSE, D), jnp.float32),
        mesh=mesh,
        compiler_params=pltpu.CompilerParams(needs_layout_passes=False,
                                             use_tc_tiling_on_sc=False),
        scratch_types=[
            pltpu.VMEM((C * NF,), jnp.float32),
            pltpu.VMEM((NGR,), jnp.int32),
            pltpu.VMEM((NGR, D), jnp.float32),
            pltpu.VMEM((NTOK, D), jnp.float32),
            pltpu.VMEM((N_SPARSE * D,), jnp.int32),
            pltpu.SemaphoreType.DMA,
        ],
    )
    def k(x_hbm, tflat_hbm, g_hbm, xbuf, idxbuf, rows, outb, vbuf, sem):
        wid = lax.axis_index("s") * NC + lax.axis_index("c")
        base_w = wid * ROWS_PER_W

        def do_chunk(c, _):
            base = base_w + c * C
            pltpu.sync_copy(x_hbm.at[pl.ds(base * NF, C * NF)], xbuf)
            bvec = lax.iota(jnp.int32, LANES)

            def build_f(f, _):
                v = plsc.load_gather(xbuf, [bvec * NF + (N_DENSE + f)])
                vi = v.astype(jnp.int32)
                vbuf[pl.ds(f * D, D)] = vi
                vg = lax.shift_right_logical(vi, 4)

                def build_d(d, _):
                    m = vg + ((f * D + d) * VGR)
                    plsc.store_scatter(idxbuf, [bvec * (N_SPARSE * D)
                                                + f * D + d], m)
                    return _

                lax.fori_loop(0, D, build_d, None, unroll=4)
                return _

            lax.fori_loop(0, N_SPARSE, build_f, None)
            pltpu.async_copy(tflat_hbm.at[idxbuf], rows, sem).wait()

            def extr_f(f, _):
                vi = vbuf[pl.ds(f * D, D)]
                col = lax.rem(vi, D)
                t16 = bvec * N_SPARSE + f

                def extr_d(d, _):
                    vals = plsc.load_gather(rows, [t16 * D + d, col])
                    plsc.store_scatter(outb, [t16, bvec * 0 + d], vals)
                    return _

                lax.fori_loop(0, D, extr_d, None, unroll=4)
                return _

            lax.fori_loop(0, N_SPARSE, extr_f, None)
            pltpu.sync_copy(outb, g_hbm.at[pl.ds(base * N_SPARSE, NTOK)])
            return _

        lax.fori_loop(0, NCH, do_chunk, None)

    return k(x_flat, tflat)


def _tc_dense_ln(x_t, g2, ew_t, b_col, sel, selt, gamma_col, beta_col):
    R = 512
    w208 = N_DENSE * D
    w624 = NF * D

    def body(x_ref, g_ref, ew_ref, b_ref, s_ref, st_ref, gm_ref, bt_ref, o_ref):
        dense = jnp.dot(ew_ref[...], x_ref[:N_DENSE, :],
                        preferred_element_type=jnp.float32) + b_ref[...]
        g_t = g_ref[...].T
        comb = jnp.concatenate([dense, g_t], axis=0)
        ssum = jnp.dot(st_ref[...], comb, preferred_element_type=jnp.float32)
        mean = ssum * (1.0 / D)
        cent = comb - jnp.dot(s_ref[...], mean,
                              preferred_element_type=jnp.float32)
        vsum = jnp.dot(st_ref[...], cent * cent,
                       preferred_element_type=jnp.float32)
        rinv = lax.rsqrt(vsum * (1.0 / D) + 1e-5)
        rb = jnp.dot(s_ref[...], rinv, preferred_element_type=jnp.float32)
        o_ref[...] = cent * rb * gm_ref[...] + bt_ref[...]

    full = lambda s: pl.BlockSpec(s, lambda i: (0, 0))
    return pl.pallas_call(
        body,
        grid=(B // R,),
        in_specs=[
            pl.BlockSpec((NF, R), lambda i: (0, i)),
            pl.BlockSpec((R, N_SPARSE * D), lambda i: (i, 0)),
            full((w208, N_DENSE)),
            full((w208, 1)),
            full((w624, NF)),
            full((NF, w624)),
            full((w624, 1)),
            full((w624, 1)),
        ],
        out_specs=pl.BlockSpec((w624, R), lambda i: (0, i)),
        out_shape=jax.ShapeDtypeStruct((w624, B), jnp.float32),
    )(x_t, g2, ew_t, b_col, sel, selt, gamma_col, beta_col)


def kernel(x, tables, lin_w, lin_b, ln_gamma, ln_beta):
    tflat = jnp.transpose(tables, (0, 2, 1)).reshape(N_SPARSE * VOCAB, D)
    g = _sc_gather(x.reshape(-1), tflat)

    ew_t = (jnp.eye(N_DENSE, dtype=jnp.float32)[:, :, None]
            * lin_w[None, :, :]).reshape(N_DENSE, N_DENSE * D).T
    b_col = lin_b.reshape(N_DENSE * D, 1)
    sel = jnp.repeat(jnp.eye(NF, dtype=jnp.float32), D, axis=0)
    selt = sel.T
    gamma_col = jnp.tile(ln_gamma, NF).reshape(NF * D, 1)
    beta_col = jnp.tile(ln_beta, NF).reshape(NF * D, 1)

    out_t = _tc_dense_ln(x.T, g.reshape(B, N_SPARSE * D),
                         ew_t, b_col, sel, selt, gamma_col, beta_col)
    return out_t.reshape(NF, D, B).transpose(2, 0, 1)
